# baseline (speedup 1.0000x reference)
import functools

import jax
import jax.numpy as jnp
from jax import lax
from jax.experimental import pallas as pl
from jax.experimental.pallas import tpu as pltpu
from jax.experimental.pallas import tpu_sc as plsc

N_NODES = 50000
N_EDGES = 800000
H = 64
NW = 32
QUARTERS = ((0, 12504), (12504, 12496), (25000, 12504), (37504, 12496))
QMAX = 12504
TRASH = 512
ACC_ROWS = QMAX + TRASH

F32 = jnp.float32
I32 = jnp.int32


def _mm_bias_relu_mm(x_ref, w1_ref, b1_ref, w2_ref, b2_ref, o_ref):
    t = jnp.maximum(jnp.dot(x_ref[...], w1_ref[...],
                            preferred_element_type=F32) + b1_ref[...], 0.0)
    o_ref[...] = jnp.dot(t, w2_ref[...], preferred_element_type=F32) + b2_ref[...]


def _tc_mlp2(x, w1, b1, w2, b2, br):
    r = x.shape[0]
    k = x.shape[1]
    hmid = w1.shape[1]
    hout = w2.shape[1]
    return pl.pallas_call(
        _mm_bias_relu_mm,
        grid=(r // br,),
        in_specs=[
            pl.BlockSpec((br, k), lambda i: (i, 0)),
            pl.BlockSpec((k, hmid), lambda i: (0, 0)),
            pl.BlockSpec((1, hmid), lambda i: (0, 0)),
            pl.BlockSpec((hmid, hout), lambda i: (0, 0)),
            pl.BlockSpec((1, hout), lambda i: (0, 0)),
        ],
        out_specs=pl.BlockSpec((br, hout), lambda i: (i, 0)),
        out_shape=jax.ShapeDtypeStruct((r, hout), F32),
    )(x, w1, b1, w2, b2)


def _mm_bias(x_ref, w_ref, b_ref, o_ref):
    o_ref[...] = jnp.dot(x_ref[...], w_ref[...],
                         preferred_element_type=F32) + b_ref[...]


def _tc_matmul(x, w, b, br):
    r = x.shape[0]
    k = x.shape[1]
    m = w.shape[1]
    return pl.pallas_call(
        _mm_bias,
        grid=(r // br,),
        in_specs=[
            pl.BlockSpec((br, k), lambda i: (i, 0)),
            pl.BlockSpec((k, m), lambda i: (0, 0)),
            pl.BlockSpec((1, m), lambda i: (0, 0)),
        ],
        out_specs=pl.BlockSpec((br, m), lambda i: (i, 0)),
        out_shape=jax.ShapeDtypeStruct((r, m), F32),
    )(x, w, b)


def _node_mm_body(h_ref, w_ref, b_ref, a1_ref, ts_ref, td_ref):
    y = jnp.dot(h_ref[...], w_ref[...], preferred_element_type=F32) + b_ref[...]
    a1_ref[...] = y[:, 0:64]
    ts_ref[...] = y[:, 64:192]
    td_ref[...] = y[:, 192:320]


def _tc_node_mm(h, wcat, bcat, br):
    r = h.shape[0]
    return pl.pallas_call(
        _node_mm_body,
        grid=(r // br,),
        in_specs=[
            pl.BlockSpec((br, 64), lambda i: (i, 0)),
            pl.BlockSpec((64, 320), lambda i: (0, 0)),
            pl.BlockSpec((1, 320), lambda i: (0, 0)),
        ],
        out_specs=[
            pl.BlockSpec((br, 64), lambda i: (i, 0)),
            pl.BlockSpec((br, 128), lambda i: (i, 0)),
            pl.BlockSpec((br, 128), lambda i: (i, 0)),
        ],
        out_shape=[
            jax.ShapeDtypeStruct((r, 64), F32),
            jax.ShapeDtypeStruct((r, 128), F32),
            jax.ShapeDtypeStruct((r, 128), F32),
        ],
    )(h, wcat, bcat)


def _edge_elem_body(gs_ref, gd_ref, b3e_ref, et_ref, mfs_ref, mbs_ref,
                    st_ref):
    i = pl.program_id(0)
    et = gs_ref[:, 0:64] + gd_ref[:, 0:64] + b3e_ref[...]
    sig = jax.nn.sigmoid(et)
    et_ref[...] = et
    mfs_ref[...] = jnp.concatenate([gs_ref[:, 64:128] * sig, sig], axis=1)
    mbs_ref[...] = jnp.concatenate([gd_ref[:, 64:128] * sig, sig], axis=1)
    s = jnp.sum(et, axis=0, keepdims=True)
    s2 = jnp.sum(et * et, axis=0, keepdims=True)
    upd = jnp.concatenate([s, s2, jnp.zeros((6, 64), F32)], axis=0)

    @pl.when(i == 0)
    def _():
        st_ref[...] = jnp.zeros_like(st_ref)

    st_ref[...] += upd


def _tc_edge_elem(g_src, g_dst, b3e, br):
    r = g_src.shape[0]
    return pl.pallas_call(
        _edge_elem_body,
        grid=(r // br,),
        in_specs=[
            pl.BlockSpec((br, 128), lambda i: (i, 0)),
            pl.BlockSpec((br, 128), lambda i: (i, 0)),
            pl.BlockSpec((br, 64), lambda i: (i, 0)),
        ],
        out_specs=[
            pl.BlockSpec((br, 64), lambda i: (i, 0)),
            pl.BlockSpec((br, 128), lambda i: (i, 0)),
            pl.BlockSpec((br, 128), lambda i: (i, 0)),
            pl.BlockSpec((8, 64), lambda i: (0, 0)),
        ],
        out_shape=[
            jax.ShapeDtypeStruct((r, 64), F32),
            jax.ShapeDtypeStruct((r, 128), F32),
            jax.ShapeDtypeStruct((r, 128), F32),
            jax.ShapeDtypeStruct((8, 64), F32),
        ],
    )(g_src, g_dst, b3e)


def _u_stats_body(a1_ref, fw_ref, bw_ref, u_ref, st_ref):
    i = pl.program_id(0)
    u = (a1_ref[...] + fw_ref[:, 0:64] / (fw_ref[:, 64:128] + 1e-6)
         + bw_ref[:, 0:64] / (bw_ref[:, 64:128] + 1e-6))
    u_ref[...] = u
    s = jnp.sum(u, axis=0, keepdims=True)
    s2 = jnp.sum(u * u, axis=0, keepdims=True)
    upd = jnp.concatenate([s, s2, jnp.zeros((6, 64), F32)], axis=0)

    @pl.when(i == 0)
    def _():
        st_ref[...] = jnp.zeros_like(st_ref)

    st_ref[...] += upd


def _tc_u_stats(a1h, agg_f, agg_b, br):
    r = a1h.shape[0]
    spec = pl.BlockSpec((br, 64), lambda i: (i, 0))
    spec2 = pl.BlockSpec((br, 128), lambda i: (i, 0))
    return pl.pallas_call(
        _u_stats_body,
        grid=(r // br,),
        in_specs=[spec, spec2, spec2],
        out_specs=[spec, pl.BlockSpec((8, 64), lambda i: (0, 0))],
        out_shape=[jax.ShapeDtypeStruct((r, 64), F32),
                   jax.ShapeDtypeStruct((8, 64), F32)],
    )(a1h, agg_f, agg_b)


def _bn_apply_body(nrows, pad_out, u_ref, st_ref, g_ref, b_ref, res_ref, o_ref):
    mean = st_ref[0:1, :] / nrows
    var = st_ref[1:2, :] / nrows - mean * mean
    bn = (u_ref[...] - mean) * lax.rsqrt(var + 1e-5) * g_ref[...] + b_ref[...]
    v = res_ref[...] + jnp.maximum(bn, 0.0)
    if pad_out:
        o_ref[...] = jnp.concatenate([v, jnp.zeros_like(v)], axis=1)
    else:
        o_ref[...] = v


def _tc_bn_apply(u, stats, gamma, beta, res, br, pad_out=False):
    r = u.shape[0]
    w = 128 if pad_out else 64
    spec = pl.BlockSpec((br, 64), lambda i: (i, 0))
    return pl.pallas_call(
        functools.partial(_bn_apply_body, float(r), pad_out),
        grid=(r // br,),
        in_specs=[spec,
                  pl.BlockSpec((8, 64), lambda i: (0, 0)),
                  pl.BlockSpec((1, 64), lambda i: (0, 0)),
                  pl.BlockSpec((1, 64), lambda i: (0, 0)),
                  spec],
        out_specs=pl.BlockSpec((br, w), lambda i: (i, 0)),
        out_shape=jax.ShapeDtypeStruct((r, w), F32),
    )(u, stats, gamma, beta, res)


def _pred_body(gs_ref, gd_ref, ef_ref, w1s_ref, w1d_ref, w1e_ref, b1_ref,
               w2_ref, b2_ref, o_ref):
    t = (jnp.dot(gs_ref[:, 0:64], w1s_ref[...], preferred_element_type=F32)
         + jnp.dot(gd_ref[:, 0:64], w1d_ref[...], preferred_element_type=F32)
         + jnp.dot(ef_ref[...], w1e_ref[...], preferred_element_type=F32)
         + b1_ref[...])
    t = jnp.maximum(t, 0.0)
    o_ref[...] = jnp.dot(t, w2_ref[...], preferred_element_type=F32) + b2_ref[...]


def _tc_predictor(gh_src, gh_dst, ef, w1s, w1d, w1e, b1, w2p, b2p, br):
    r = gh_src.shape[0]
    spec = pl.BlockSpec((br, 64), lambda i: (i, 0))
    spec2 = pl.BlockSpec((br, 128), lambda i: (i, 0))
    wspec = pl.BlockSpec((64, 64), lambda i: (0, 0))
    return pl.pallas_call(
        _pred_body,
        grid=(r // br,),
        in_specs=[spec2, spec2, spec, wspec, wspec, wspec,
                  pl.BlockSpec((1, 64), lambda i: (0, 0)),
                  pl.BlockSpec((64, 8), lambda i: (0, 0)),
                  pl.BlockSpec((1, 8), lambda i: (0, 0))],
        out_specs=pl.BlockSpec((br, 8), lambda i: (i, 0)),
        out_shape=jax.ShapeDtypeStruct((r, 8), F32),
    )(gh_src, gh_dst, ef, w1s, w1d, w1e, b1, w2p, b2p)


def _sc_gather2(table1, idx1, table2, idx2, d):
    e = idx1.shape[0]
    per_w = e // NW
    c = 200
    nchunk = per_w // c
    mesh = plsc.VectorSubcoreMesh(core_axis_name="c", subcore_axis_name="s")

    @functools.partial(
        pl.kernel, mesh=mesh,
        out_type=[jax.ShapeDtypeStruct((e, d), F32),
                  jax.ShapeDtypeStruct((e, d), F32)],
        scratch_types=[pltpu.VMEM((c,), I32),
                       pltpu.VMEM((c,), I32),
                       pltpu.VMEM((c, d), F32),
                       pltpu.VMEM((c, d), F32),
                       pltpu.SemaphoreType.DMA,
                       pltpu.SemaphoreType.DMA],
    )
    def k(t1_hbm, i1_hbm, t2_hbm, i2_hbm, o1_hbm, o2_hbm, i0, i1, r0, r1,
          s0, s1):
        wid = lax.axis_index("s") * 2 + lax.axis_index("c")
        w0 = wid * per_w
        bufs = ((i0, r0, s0), (i1, r1, s1))

        for table_hbm, idx_hbm, out_hbm in ((t1_hbm, i1_hbm, o1_hbm),
                                            (t2_hbm, i2_hbm, o2_hbm)):
            for b, (ib, rb, sb) in enumerate(bufs):
                pltpu.sync_copy(idx_hbm.at[pl.ds(w0 + b * c, c)], ib)
                pltpu.async_copy(table_hbm.at[ib], rb, sb)

            def body(j, carry):
                for b, (ib, rb, sb) in enumerate(bufs):
                    chunk = 2 * j + b
                    pltpu.make_async_copy(table_hbm.at[ib], rb, sb).wait()
                    pltpu.sync_copy(rb, out_hbm.at[pl.ds(w0 + chunk * c, c)])
                    nxt = chunk + 2

                    @pl.when(nxt < nchunk)
                    def _():
                        pltpu.sync_copy(idx_hbm.at[pl.ds(w0 + nxt * c, c)], ib)
                        pltpu.async_copy(table_hbm.at[ib], rb, sb)

                return carry

            lax.fori_loop(0, nchunk // 2, body, 0)

            if nchunk % 2 == 1:
                ib, rb, sb = bufs[(nchunk - 1) % 2]
                pltpu.make_async_copy(table_hbm.at[ib], rb, sb).wait()
                pltpu.sync_copy(rb, out_hbm.at[pl.ds(w0 + (nchunk - 1) * c, c)])

    return k(table1, idx1, table2, idx2)


def _sc_scatter_add(values, idx, zeros_hbm):
    e = idx.shape[0]
    c = 80
    nchunks = e // c
    trips = nchunks // 16
    mesh = plsc.VectorSubcoreMesh(core_axis_name="c", subcore_axis_name="s")

    @functools.partial(
        pl.kernel, mesh=mesh,
        out_type=jax.ShapeDtypeStruct((N_NODES, 2 * H), F32),
        scratch_types=[pltpu.VMEM_SHARED((ACC_ROWS, 2 * H), F32),
                       pltpu.VMEM((c,), I32),
                       pltpu.VMEM((c,), I32),
                       pltpu.VMEM((c, 2 * H), F32),
                       pltpu.VMEM((c, 2 * H), F32),
                       pltpu.SemaphoreType.DMA,
                       pltpu.SemaphoreType.DMA,
                       pltpu.SemaphoreType.DMA,
                       pltpu.SemaphoreType.DMA],
    )
    def k(val_hbm, idx_hbm, z_hbm, out_hbm, acc, i0, i1, v0, v1, s0, s1,
          t0, t1):
        cid = lax.axis_index("c")
        sid = lax.axis_index("s")
        bufs = ((i0, v0, s0, t0), (i1, v1, s1, t1))

        def start_loads(t, b):
            ib, vb, sb, tb = bufs[b]
            base = (sid + t * 16) * c

            @pl.when(t >= 2)
            def _():
                pltpu.make_async_copy(vb, acc.at[ib], tb).wait()

            pltpu.async_copy(idx_hbm.at[pl.ds(base, c)], ib, sb)
            pltpu.async_copy(val_hbm.at[pl.ds(base, c)], vb, sb)

        def wait_loads(t, b):
            ib, vb, sb, tb = bufs[b]
            base = (sid + t * 16) * c
            pltpu.make_async_copy(idx_hbm.at[pl.ds(base, c)], ib, sb).wait()
            pltpu.make_async_copy(val_hbm.at[pl.ds(base, c)], vb, sb).wait()

        for p in range(2):
            qb = jnp.where(cid == 0, QUARTERS[2 * p][0], QUARTERS[2 * p + 1][0])
            qs = jnp.where(cid == 0, QUARTERS[2 * p][1], QUARTERS[2 * p + 1][1])

            @pl.when(sid == 0)
            def _():
                pltpu.sync_copy(z_hbm, acc)

            plsc.subcore_barrier()
            start_loads(0, 0)

            def proc(t, b):
                ib, vb, _, tb = bufs[b]
                wait_loads(t, b)

                @pl.when(t + 1 < trips)
                def _():
                    start_loads(t + 1, b ^ 1)

                def vec_body(kk, carry2):
                    v = ib[pl.ds(kk * 16, 16)]
                    rel = v - qb
                    ok = (rel >= 0) & (rel < qs)
                    lane = lax.iota(I32, 16)
                    trash = QMAX + ((kk * 16 + lane + sid * 97) & (TRASH - 1))
                    ib[pl.ds(kk * 16, 16)] = jnp.where(ok, rel, trash)
                    return carry2

                lax.fori_loop(0, c // 16, vec_body, 0)
                pltpu.async_copy(vb, acc.at[ib], tb, add=True)

            def pair_body(jp, carry):
                for b in range(2):
                    t = 2 * jp + b

                    @pl.when(t < trips)
                    def _():
                        proc(t, b)

                return carry

            lax.fori_loop(0, (trips + 1) // 2, pair_body, 0)
            for b in range(2):
                ib, vb, _, tb = bufs[b]

                @pl.when(trips >= 2 - b)
                def _():
                    pltpu.make_async_copy(vb, acc.at[ib], tb).wait()

            plsc.subcore_barrier()

            r0 = 776
            pltpu.sync_copy(acc.at[pl.ds(sid * r0, r0)],
                            out_hbm.at[pl.ds(qb + sid * r0, r0)])

            @pl.when(sid == 0)
            def _():
                pltpu.sync_copy(acc.at[pl.ds(16 * r0, 80)],
                                out_hbm.at[pl.ds(qb + 16 * r0, 80)])

            @pl.when((sid == 1) & (cid == 0))
            def _():
                pltpu.sync_copy(acc.at[pl.ds(12496, 8)],
                                out_hbm.at[pl.ds(qb + 12496, 8)])

            if p == 0:
                plsc.subcore_barrier()

    return k(values, idx, zeros_hbm)


def kernel(x, e, edge_index, Wn1, bn1, Wn2, bn2, We, be, A1, A1b, A2, A2b,
           A3, A3b, B1, B1b, B2, B2b, B3, B3b, gn, gnb, ge, geb, Wp1, bp1,
           Wp2, bp2):
    src = edge_index[0].astype(I32)
    dst = edge_index[1].astype(I32)

    xp = jnp.pad(x, ((0, 0), (0, 6)))
    ep = jnp.pad(e, ((0, 0), (0, 6)))
    wn1p = jnp.pad(Wn1, ((0, 6), (0, 0)))
    wep = jnp.pad(We, ((0, 6), (0, 0)))

    h = _tc_mlp2(xp, wn1p, bn1[None, :], Wn2, bn2[None, :], 1000)
    ef = _tc_matmul(ep, wep, be[None, :], 8000)

    zeros_hbm = jnp.zeros((ACC_ROWS, 2 * H), F32)

    for l in range(3):
        wcat = jnp.concatenate([A1[l], B1[l], A2[l], B2[l], A3[l]], axis=1)
        bcat = jnp.concatenate([A1b[l], B1b[l], A2b[l], B2b[l], A3b[l]],
                               axis=0)[None, :]
        a1h, t_src, t_dst = _tc_node_mm(h, wcat, bcat, 1000)
        b3e = _tc_matmul(ef, B3[l], B3b[l][None, :], 8000)

        g_src, g_dst = _sc_gather2(t_src, src, t_dst, dst, 128)

        e_tmp, mfsig, mbsig, est = _tc_edge_elem(g_src, g_dst, b3e, 8000)

        agg_f = _sc_scatter_add(mfsig, dst, zeros_hbm)
        agg_b = _sc_scatter_add(mbsig, src, zeros_hbm)

        u, ust = _tc_u_stats(a1h, agg_f, agg_b, 1000)
        h = _tc_bn_apply(u, ust, gn[l][None, :], gnb[l][None, :], h, 1000,
                         pad_out=(l == 2))
        ef = _tc_bn_apply(e_tmp, est, ge[l][None, :], geb[l][None, :], ef, 8000)

    gh_src, gh_dst = _sc_gather2(h, src, h, dst, 128)

    w2p = jnp.pad(Wp2, ((0, 0), (0, 7)))
    b2p = jnp.pad(bp2, ((0, 7)))[None, :]
    s8 = _tc_predictor(gh_src, gh_dst, ef, Wp1[0:64], Wp1[64:128],
                       Wp1[128:192], bp1[None, :], w2p, b2p, 8000)
    return s8[:, 0:1]


# speedup vs baseline: 1.0456x; 1.0456x over previous
import functools

import jax
import jax.numpy as jnp
from jax import lax
from jax.experimental import pallas as pl
from jax.experimental.pallas import tpu as pltpu
from jax.experimental.pallas import tpu_sc as plsc

N_NODES = 50000
N_EDGES = 800000
H = 64
NW = 32
QUARTERS = ((0, 12504), (12504, 12496), (25000, 12504), (37504, 12496))
QMAX = 12504
TRASH = 512
ACC_ROWS = QMAX + TRASH

F32 = jnp.float32
I32 = jnp.int32


def _mm_bias_relu_mm(x_ref, w1_ref, b1_ref, w2_ref, b2_ref, o_ref):
    t = jnp.maximum(jnp.dot(x_ref[...], w1_ref[...],
                            preferred_element_type=F32) + b1_ref[...], 0.0)
    o_ref[...] = jnp.dot(t, w2_ref[...], preferred_element_type=F32) + b2_ref[...]


def _tc_mlp2(x, w1, b1, w2, b2, br):
    r = x.shape[0]
    k = x.shape[1]
    hmid = w1.shape[1]
    hout = w2.shape[1]
    return pl.pallas_call(
        _mm_bias_relu_mm,
        grid=(r // br,),
        in_specs=[
            pl.BlockSpec((br, k), lambda i: (i, 0)),
            pl.BlockSpec((k, hmid), lambda i: (0, 0)),
            pl.BlockSpec((1, hmid), lambda i: (0, 0)),
            pl.BlockSpec((hmid, hout), lambda i: (0, 0)),
            pl.BlockSpec((1, hout), lambda i: (0, 0)),
        ],
        out_specs=pl.BlockSpec((br, hout), lambda i: (i, 0)),
        out_shape=jax.ShapeDtypeStruct((r, hout), F32),
    )(x, w1, b1, w2, b2)


def _mm_bias(x_ref, w_ref, b_ref, o_ref):
    o_ref[...] = jnp.dot(x_ref[...], w_ref[...],
                         preferred_element_type=F32) + b_ref[...]


def _tc_matmul(x, w, b, br):
    r = x.shape[0]
    k = x.shape[1]
    m = w.shape[1]
    return pl.pallas_call(
        _mm_bias,
        grid=(r // br,),
        in_specs=[
            pl.BlockSpec((br, k), lambda i: (i, 0)),
            pl.BlockSpec((k, m), lambda i: (0, 0)),
            pl.BlockSpec((1, m), lambda i: (0, 0)),
        ],
        out_specs=pl.BlockSpec((br, m), lambda i: (i, 0)),
        out_shape=jax.ShapeDtypeStruct((r, m), F32),
    )(x, w, b)


def _node_mm_body(h_ref, w_ref, b_ref, a1_ref, ts_ref, td_ref):
    y = jnp.dot(h_ref[...], w_ref[...], preferred_element_type=F32) + b_ref[...]
    a1_ref[...] = y[:, 0:64]
    ts_ref[...] = y[:, 64:192]
    td_ref[...] = y[:, 192:320]


def _tc_node_mm(h, wcat, bcat, br):
    r = h.shape[0]
    return pl.pallas_call(
        _node_mm_body,
        grid=(r // br,),
        in_specs=[
            pl.BlockSpec((br, 64), lambda i: (i, 0)),
            pl.BlockSpec((64, 320), lambda i: (0, 0)),
            pl.BlockSpec((1, 320), lambda i: (0, 0)),
        ],
        out_specs=[
            pl.BlockSpec((br, 64), lambda i: (i, 0)),
            pl.BlockSpec((br, 128), lambda i: (i, 0)),
            pl.BlockSpec((br, 128), lambda i: (i, 0)),
        ],
        out_shape=[
            jax.ShapeDtypeStruct((r, 64), F32),
            jax.ShapeDtypeStruct((r, 128), F32),
            jax.ShapeDtypeStruct((r, 128), F32),
        ],
    )(h, wcat, bcat)


def _edge_elem_body(gs_ref, gd_ref, ef_ref, w3_ref, b3_ref, et_ref,
                    mfs_ref, mbs_ref, st_ref):
    i = pl.program_id(0)
    b3e = jnp.dot(ef_ref[...], w3_ref[...],
                  preferred_element_type=F32) + b3_ref[...]
    et = gs_ref[:, 0:64] + gd_ref[:, 0:64] + b3e
    sig = jax.nn.sigmoid(et)
    et_ref[...] = et
    mfs_ref[...] = jnp.concatenate([gs_ref[:, 64:128] * sig, sig], axis=1)
    mbs_ref[...] = jnp.concatenate([gd_ref[:, 64:128] * sig, sig], axis=1)
    s = jnp.sum(et, axis=0, keepdims=True)
    s2 = jnp.sum(et * et, axis=0, keepdims=True)
    upd = jnp.concatenate([s, s2, jnp.zeros((6, 64), F32)], axis=0)

    @pl.when(i == 0)
    def _():
        st_ref[...] = jnp.zeros_like(st_ref)

    st_ref[...] += upd


def _tc_edge_elem(g_src, g_dst, ef, w3, b3, br):
    r = g_src.shape[0]
    return pl.pallas_call(
        _edge_elem_body,
        grid=(r // br,),
        in_specs=[
            pl.BlockSpec((br, 128), lambda i: (i, 0)),
            pl.BlockSpec((br, 128), lambda i: (i, 0)),
            pl.BlockSpec((br, 64), lambda i: (i, 0)),
            pl.BlockSpec((64, 64), lambda i: (0, 0)),
            pl.BlockSpec((1, 64), lambda i: (0, 0)),
        ],
        out_specs=[
            pl.BlockSpec((br, 64), lambda i: (i, 0)),
            pl.BlockSpec((br, 128), lambda i: (i, 0)),
            pl.BlockSpec((br, 128), lambda i: (i, 0)),
            pl.BlockSpec((8, 64), lambda i: (0, 0)),
        ],
        out_shape=[
            jax.ShapeDtypeStruct((r, 64), F32),
            jax.ShapeDtypeStruct((r, 128), F32),
            jax.ShapeDtypeStruct((r, 128), F32),
            jax.ShapeDtypeStruct((8, 64), F32),
        ],
    )(g_src, g_dst, ef, w3, b3)


def _u_stats_body(a1_ref, fw_ref, bw_ref, u_ref, st_ref):
    i = pl.program_id(0)
    u = (a1_ref[...] + fw_ref[:, 0:64] / (fw_ref[:, 64:128] + 1e-6)
         + bw_ref[:, 0:64] / (bw_ref[:, 64:128] + 1e-6))
    u_ref[...] = u
    s = jnp.sum(u, axis=0, keepdims=True)
    s2 = jnp.sum(u * u, axis=0, keepdims=True)
    upd = jnp.concatenate([s, s2, jnp.zeros((6, 64), F32)], axis=0)

    @pl.when(i == 0)
    def _():
        st_ref[...] = jnp.zeros_like(st_ref)

    st_ref[...] += upd


def _tc_u_stats(a1h, agg_f, agg_b, br):
    r = a1h.shape[0]
    spec = pl.BlockSpec((br, 64), lambda i: (i, 0))
    spec2 = pl.BlockSpec((br, 128), lambda i: (i, 0))
    return pl.pallas_call(
        _u_stats_body,
        grid=(r // br,),
        in_specs=[spec, spec2, spec2],
        out_specs=[spec, pl.BlockSpec((8, 64), lambda i: (0, 0))],
        out_shape=[jax.ShapeDtypeStruct((r, 64), F32),
                   jax.ShapeDtypeStruct((8, 64), F32)],
    )(a1h, agg_f, agg_b)


def _bn_apply_body(nrows, pad_out, u_ref, st_ref, g_ref, b_ref, res_ref, o_ref):
    mean = st_ref[0:1, :] / nrows
    var = st_ref[1:2, :] / nrows - mean * mean
    bn = (u_ref[...] - mean) * lax.rsqrt(var + 1e-5) * g_ref[...] + b_ref[...]
    v = res_ref[...] + jnp.maximum(bn, 0.0)
    if pad_out:
        o_ref[...] = jnp.concatenate([v, jnp.zeros_like(v)], axis=1)
    else:
        o_ref[...] = v


def _tc_bn_apply(u, stats, gamma, beta, res, br, pad_out=False):
    r = u.shape[0]
    w = 128 if pad_out else 64
    spec = pl.BlockSpec((br, 64), lambda i: (i, 0))
    return pl.pallas_call(
        functools.partial(_bn_apply_body, float(r), pad_out),
        grid=(r // br,),
        in_specs=[spec,
                  pl.BlockSpec((8, 64), lambda i: (0, 0)),
                  pl.BlockSpec((1, 64), lambda i: (0, 0)),
                  pl.BlockSpec((1, 64), lambda i: (0, 0)),
                  spec],
        out_specs=pl.BlockSpec((br, w), lambda i: (i, 0)),
        out_shape=jax.ShapeDtypeStruct((r, w), F32),
    )(u, stats, gamma, beta, res)


def _pred_body(gs_ref, gd_ref, ef_ref, w1s_ref, w1d_ref, w1e_ref, b1_ref,
               w2_ref, b2_ref, o_ref):
    t = (jnp.dot(gs_ref[:, 0:64], w1s_ref[...], preferred_element_type=F32)
         + jnp.dot(gd_ref[:, 0:64], w1d_ref[...], preferred_element_type=F32)
         + jnp.dot(ef_ref[...], w1e_ref[...], preferred_element_type=F32)
         + b1_ref[...])
    t = jnp.maximum(t, 0.0)
    o_ref[...] = jnp.dot(t, w2_ref[...], preferred_element_type=F32) + b2_ref[...]


def _tc_predictor(gh_src, gh_dst, ef, w1s, w1d, w1e, b1, w2p, b2p, br):
    r = gh_src.shape[0]
    spec = pl.BlockSpec((br, 64), lambda i: (i, 0))
    spec2 = pl.BlockSpec((br, 128), lambda i: (i, 0))
    wspec = pl.BlockSpec((64, 64), lambda i: (0, 0))
    return pl.pallas_call(
        _pred_body,
        grid=(r // br,),
        in_specs=[spec2, spec2, spec, wspec, wspec, wspec,
                  pl.BlockSpec((1, 64), lambda i: (0, 0)),
                  pl.BlockSpec((64, 8), lambda i: (0, 0)),
                  pl.BlockSpec((1, 8), lambda i: (0, 0))],
        out_specs=pl.BlockSpec((br, 8), lambda i: (i, 0)),
        out_shape=jax.ShapeDtypeStruct((r, 8), F32),
    )(gh_src, gh_dst, ef, w1s, w1d, w1e, b1, w2p, b2p)


def _sc_gather2(table1, idx1, table2, idx2, d):
    e = idx1.shape[0]
    per_w = e // NW
    c = 200
    nchunk = per_w // c
    mesh = plsc.VectorSubcoreMesh(core_axis_name="c", subcore_axis_name="s")

    @functools.partial(
        pl.kernel, mesh=mesh,
        out_type=[jax.ShapeDtypeStruct((e, d), F32),
                  jax.ShapeDtypeStruct((e, d), F32)],
        scratch_types=[pltpu.VMEM((c,), I32),
                       pltpu.VMEM((c,), I32),
                       pltpu.VMEM((c, d), F32),
                       pltpu.VMEM((c, d), F32),
                       pltpu.SemaphoreType.DMA,
                       pltpu.SemaphoreType.DMA],
    )
    def k(t1_hbm, i1_hbm, t2_hbm, i2_hbm, o1_hbm, o2_hbm, i0, i1, r0, r1,
          s0, s1):
        wid = lax.axis_index("s") * 2 + lax.axis_index("c")
        w0 = wid * per_w
        bufs = ((i0, r0, s0), (i1, r1, s1))

        for table_hbm, idx_hbm, out_hbm in ((t1_hbm, i1_hbm, o1_hbm),
                                            (t2_hbm, i2_hbm, o2_hbm)):
            for b, (ib, rb, sb) in enumerate(bufs):
                pltpu.sync_copy(idx_hbm.at[pl.ds(w0 + b * c, c)], ib)
                pltpu.async_copy(table_hbm.at[ib], rb, sb)

            def body(j, carry):
                for b, (ib, rb, sb) in enumerate(bufs):
                    chunk = 2 * j + b
                    pltpu.make_async_copy(table_hbm.at[ib], rb, sb).wait()
                    pltpu.sync_copy(rb, out_hbm.at[pl.ds(w0 + chunk * c, c)])
                    nxt = chunk + 2

                    @pl.when(nxt < nchunk)
                    def _():
                        pltpu.sync_copy(idx_hbm.at[pl.ds(w0 + nxt * c, c)], ib)
                        pltpu.async_copy(table_hbm.at[ib], rb, sb)

                return carry

            lax.fori_loop(0, nchunk // 2, body, 0)

            if nchunk % 2 == 1:
                ib, rb, sb = bufs[(nchunk - 1) % 2]
                pltpu.make_async_copy(table_hbm.at[ib], rb, sb).wait()
                pltpu.sync_copy(rb, out_hbm.at[pl.ds(w0 + (nchunk - 1) * c, c)])

    return k(table1, idx1, table2, idx2)


def _sc_scatter_add(values, idx, zeros_hbm):
    e = idx.shape[0]
    c = 80
    nchunks = e // c
    trips = nchunks // 16
    mesh = plsc.VectorSubcoreMesh(core_axis_name="c", subcore_axis_name="s")

    @functools.partial(
        pl.kernel, mesh=mesh,
        out_type=jax.ShapeDtypeStruct((N_NODES, 2 * H), F32),
        scratch_types=[pltpu.VMEM_SHARED((ACC_ROWS, 2 * H), F32),
                       pltpu.VMEM((c,), I32),
                       pltpu.VMEM((c,), I32),
                       pltpu.VMEM((c, 2 * H), F32),
                       pltpu.VMEM((c, 2 * H), F32),
                       pltpu.SemaphoreType.DMA,
                       pltpu.SemaphoreType.DMA,
                       pltpu.SemaphoreType.DMA,
                       pltpu.SemaphoreType.DMA],
    )
    def k(val_hbm, idx_hbm, z_hbm, out_hbm, acc, i0, i1, v0, v1, s0, s1,
          t0, t1):
        cid = lax.axis_index("c")
        sid = lax.axis_index("s")
        bufs = ((i0, v0, s0, t0), (i1, v1, s1, t1))

        def start_loads(t, b):
            ib, vb, sb, tb = bufs[b]
            base = (sid + t * 16) * c

            @pl.when(t >= 2)
            def _():
                pltpu.make_async_copy(vb, acc.at[ib], tb).wait()

            pltpu.async_copy(idx_hbm.at[pl.ds(base, c)], ib, sb)
            pltpu.async_copy(val_hbm.at[pl.ds(base, c)], vb, sb)

        def wait_loads(t, b):
            ib, vb, sb, tb = bufs[b]
            base = (sid + t * 16) * c
            pltpu.make_async_copy(idx_hbm.at[pl.ds(base, c)], ib, sb).wait()
            pltpu.make_async_copy(val_hbm.at[pl.ds(base, c)], vb, sb).wait()

        for p in range(2):
            qb = jnp.where(cid == 0, QUARTERS[2 * p][0], QUARTERS[2 * p + 1][0])
            qs = jnp.where(cid == 0, QUARTERS[2 * p][1], QUARTERS[2 * p + 1][1])

            @pl.when(sid == 0)
            def _():
                pltpu.sync_copy(z_hbm, acc)

            plsc.subcore_barrier()
            start_loads(0, 0)

            def proc(t, b):
                ib, vb, _, tb = bufs[b]
                wait_loads(t, b)

                @pl.when(t + 1 < trips)
                def _():
                    start_loads(t + 1, b ^ 1)

                def vec_body(kk, carry2):
                    v = ib[pl.ds(kk * 16, 16)]
                    rel = v - qb
                    ok = (rel >= 0) & (rel < qs)
                    lane = lax.iota(I32, 16)
                    trash = QMAX + ((kk * 16 + lane + sid * 97) & (TRASH - 1))
                    ib[pl.ds(kk * 16, 16)] = jnp.where(ok, rel, trash)
                    return carry2

                lax.fori_loop(0, c // 16, vec_body, 0)
                pltpu.async_copy(vb, acc.at[ib], tb, add=True)

            def pair_body(jp, carry):
                for b in range(2):
                    t = 2 * jp + b

                    @pl.when(t < trips)
                    def _():
                        proc(t, b)

                return carry

            lax.fori_loop(0, (trips + 1) // 2, pair_body, 0)
            for b in range(2):
                ib, vb, _, tb = bufs[b]

                @pl.when(trips >= 2 - b)
                def _():
                    pltpu.make_async_copy(vb, acc.at[ib], tb).wait()

            plsc.subcore_barrier()

            r0 = 776
            pltpu.sync_copy(acc.at[pl.ds(sid * r0, r0)],
                            out_hbm.at[pl.ds(qb + sid * r0, r0)])

            @pl.when(sid == 0)
            def _():
                pltpu.sync_copy(acc.at[pl.ds(16 * r0, 80)],
                                out_hbm.at[pl.ds(qb + 16 * r0, 80)])

            @pl.when((sid == 1) & (cid == 0))
            def _():
                pltpu.sync_copy(acc.at[pl.ds(12496, 8)],
                                out_hbm.at[pl.ds(qb + 12496, 8)])

            if p == 0:
                plsc.subcore_barrier()

    return k(values, idx, zeros_hbm)


def kernel(x, e, edge_index, Wn1, bn1, Wn2, bn2, We, be, A1, A1b, A2, A2b,
           A3, A3b, B1, B1b, B2, B2b, B3, B3b, gn, gnb, ge, geb, Wp1, bp1,
           Wp2, bp2):
    src = edge_index[0].astype(I32)
    dst = edge_index[1].astype(I32)

    xp = jnp.pad(x, ((0, 0), (0, 6)))
    ep = jnp.pad(e, ((0, 0), (0, 6)))
    wn1p = jnp.pad(Wn1, ((0, 6), (0, 0)))
    wep = jnp.pad(We, ((0, 6), (0, 0)))

    h = _tc_mlp2(xp, wn1p, bn1[None, :], Wn2, bn2[None, :], 1000)
    ef = _tc_matmul(ep, wep, be[None, :], 8000)

    zeros_hbm = jnp.zeros((ACC_ROWS, 2 * H), F32)

    for l in range(3):
        wcat = jnp.concatenate([A1[l], B1[l], A2[l], B2[l], A3[l]], axis=1)
        bcat = jnp.concatenate([A1b[l], B1b[l], A2b[l], B2b[l], A3b[l]],
                               axis=0)[None, :]
        a1h, t_src, t_dst = _tc_node_mm(h, wcat, bcat, 1000)

        g_src, g_dst = _sc_gather2(t_src, src, t_dst, dst, 128)

        e_tmp, mfsig, mbsig, est = _tc_edge_elem(g_src, g_dst, ef, B3[l],
                                                 B3b[l][None, :], 8000)

        agg_f = _sc_scatter_add(mfsig, dst, zeros_hbm)
        agg_b = _sc_scatter_add(mbsig, src, zeros_hbm)

        u, ust = _tc_u_stats(a1h, agg_f, agg_b, 1000)
        h = _tc_bn_apply(u, ust, gn[l][None, :], gnb[l][None, :], h, 1000,
                         pad_out=(l == 2))
        ef = _tc_bn_apply(e_tmp, est, ge[l][None, :], geb[l][None, :], ef, 8000)

    gh_src, gh_dst = _sc_gather2(h, src, h, dst, 128)

    w2p = jnp.pad(Wp2, ((0, 0), (0, 7)))
    b2p = jnp.pad(bp2, ((0, 7)))[None, :]
    s8 = _tc_predictor(gh_src, gh_dst, ef, Wp1[0:64], Wp1[64:128],
                       Wp1[128:192], bp1[None, :], w2p, b2p, 8000)
    return s8[:, 0:1]
